# baseline (device time: 29409 ns/iter reference)
import jax
import jax.numpy as jnp
from jax import lax
from jax.experimental import pallas as pl
from jax.experimental.pallas import tpu as pltpu

N_DEV = 4


def kernel(partial, gamma):
    x = partial.reshape(partial.shape[1], partial.shape[2])
    g = gamma.reshape(1, -1)
    m_tot, d = x.shape
    m_per = m_tot // N_DEV

    def body(x_ref, g_ref, out_ref, comm_ref, send_sems, recv_sems):
        my = lax.axis_index("i")
        left = lax.rem(my + N_DEV - 1, N_DEV)
        right = lax.rem(my + 1, N_DEV)

        barrier_sem = pltpu.get_barrier_semaphore()
        for nbr in (left, right):
            pl.semaphore_signal(
                barrier_sem, inc=1,
                device_id=(nbr,), device_id_type=pl.DeviceIdType.MESH,
            )
        pl.semaphore_wait(barrier_sem, 2)

        c0 = lax.rem(my + N_DEV - 1, N_DEV)
        comm_ref[0, :, :] = x_ref[pl.ds(c0 * m_per, m_per), :].astype(
            jnp.bfloat16
        )

        for s in range(N_DEV - 1):
            rdma = pltpu.make_async_remote_copy(
                src_ref=comm_ref.at[s],
                dst_ref=comm_ref.at[s + 1],
                send_sem=send_sems.at[s],
                recv_sem=recv_sems.at[s],
                device_id=(right,),
                device_id_type=pl.DeviceIdType.MESH,
            )
            rdma.start()
            rdma.wait()

            c = lax.rem(my + 2 * N_DEV - 2 - s, N_DEV)
            if s < N_DEV - 2:
                comm_ref[s + 1, :, :] = comm_ref[s + 1, :, :] + x_ref[
                    pl.ds(c * m_per, m_per), :
                ].astype(jnp.bfloat16)
            else:
                y = comm_ref[s + 1, :, :].astype(jnp.float32) + x_ref[
                    pl.ds(c * m_per, m_per), :
                ]
                rms = jnp.sqrt(jnp.mean(y * y, axis=-1, keepdims=True) + 1e-6)
                out_ref[:, :] = y / rms * g_ref[0, :]

    return pl.pallas_call(
        body,
        out_shape=jax.ShapeDtypeStruct((m_per, d), jnp.float32),
        in_specs=[
            pl.BlockSpec(memory_space=pltpu.VMEM),
            pl.BlockSpec(memory_space=pltpu.VMEM),
        ],
        out_specs=pl.BlockSpec(memory_space=pltpu.VMEM),
        scratch_shapes=[
            pltpu.VMEM((N_DEV, m_per, d), jnp.bfloat16),
            pltpu.SemaphoreType.DMA((N_DEV - 1,)),
            pltpu.SemaphoreType.DMA((N_DEV - 1,)),
        ],
        compiler_params=pltpu.CompilerParams(collective_id=0),
    )(x, g)


# device time: 19004 ns/iter; 1.5475x vs baseline; 1.5475x over previous
import jax
import jax.numpy as jnp
from jax import lax
from jax.experimental import pallas as pl
from jax.experimental.pallas import tpu as pltpu

N_DEV = 4


def kernel(partial, gamma):
    x = partial.reshape(partial.shape[1], partial.shape[2])
    g = gamma.reshape(1, -1)
    m_tot, d = x.shape
    m_per = m_tot // N_DEV
    dh = d // 2

    def body(
        x_ref, g_ref, out_ref,
        send_a1, send_b1, recv_a1, recv_b1,
        send_a2, send_b2, recv_a2, recv_b2,
        loc_a2, loc_b2,
        send_sems, recv_sems,
    ):
        my = lax.axis_index("i")
        q1 = my ^ 1
        q3 = my ^ 3
        c1 = my ^ 1
        c2 = my ^ 2
        c3 = my ^ 3

        def row(c):
            return pl.ds(c * m_per, m_per)

        send_a1[0, :, :] = x_ref[row(c1), :dh].astype(jnp.bfloat16)
        send_a1[1, :, :] = x_ref[row(c2), :dh].astype(jnp.bfloat16)
        send_b1[0, :, :] = x_ref[row(c3), dh:].astype(jnp.bfloat16)
        send_b1[1, :, :] = x_ref[row(c2), dh:].astype(jnp.bfloat16)
        loc_a2[:, :] = x_ref[row(c3), :dh].astype(jnp.bfloat16)
        loc_b2[:, :] = x_ref[row(c1), dh:].astype(jnp.bfloat16)

        barrier_sem = pltpu.get_barrier_semaphore()
        for nbr in (q1, q3):
            pl.semaphore_signal(
                barrier_sem, inc=1,
                device_id=(nbr,), device_id_type=pl.DeviceIdType.MESH,
            )
        pl.semaphore_wait(barrier_sem, 2)

        rdma_a1 = pltpu.make_async_remote_copy(
            src_ref=send_a1, dst_ref=recv_a1,
            send_sem=send_sems.at[0], recv_sem=recv_sems.at[0],
            device_id=(q1,), device_id_type=pl.DeviceIdType.MESH,
        )
        rdma_b1 = pltpu.make_async_remote_copy(
            src_ref=send_b1, dst_ref=recv_b1,
            send_sem=send_sems.at[1], recv_sem=recv_sems.at[1],
            device_id=(q3,), device_id_type=pl.DeviceIdType.MESH,
        )
        rdma_a1.start()
        rdma_b1.start()

        rdma_a1.wait_recv()
        send_a2[:, :] = loc_a2[:, :] + recv_a1[1, :, :]
        rdma_a2 = pltpu.make_async_remote_copy(
            src_ref=send_a2, dst_ref=recv_a2,
            send_sem=send_sems.at[2], recv_sem=recv_sems.at[2],
            device_id=(q3,), device_id_type=pl.DeviceIdType.MESH,
        )
        rdma_a2.start()

        rdma_b1.wait_recv()
        send_b2[:, :] = loc_b2[:, :] + recv_b1[1, :, :]
        rdma_b2 = pltpu.make_async_remote_copy(
            src_ref=send_b2, dst_ref=recv_b2,
            send_sem=send_sems.at[3], recv_sem=recv_sems.at[3],
            device_id=(q1,), device_id_type=pl.DeviceIdType.MESH,
        )
        rdma_b2.start()

        rdma_a2.wait_recv()
        rdma_b2.wait_recv()
        y_a = (
            x_ref[row(my), :dh]
            + recv_a1[0, :, :].astype(jnp.float32)
            + recv_a2[:, :].astype(jnp.float32)
        )
        y_b = (
            x_ref[row(my), dh:]
            + recv_b1[0, :, :].astype(jnp.float32)
            + recv_b2[:, :].astype(jnp.float32)
        )
        ss = jnp.sum(y_a * y_a, axis=-1, keepdims=True) + jnp.sum(
            y_b * y_b, axis=-1, keepdims=True
        )
        inv_rms = lax.rsqrt(ss / d + 1e-6)
        out_ref[:, :dh] = y_a * inv_rms * g_ref[0, :dh]
        out_ref[:, dh:] = y_b * inv_rms * g_ref[0, dh:]

        rdma_a1.wait_send()
        rdma_b1.wait_send()
        rdma_a2.wait_send()
        rdma_b2.wait_send()

    bf = jnp.bfloat16
    return pl.pallas_call(
        body,
        out_shape=jax.ShapeDtypeStruct((m_per, d), jnp.float32),
        in_specs=[
            pl.BlockSpec(memory_space=pltpu.VMEM),
            pl.BlockSpec(memory_space=pltpu.VMEM),
        ],
        out_specs=pl.BlockSpec(memory_space=pltpu.VMEM),
        scratch_shapes=[
            pltpu.VMEM((2, m_per, dh), bf),
            pltpu.VMEM((2, m_per, dh), bf),
            pltpu.VMEM((2, m_per, dh), bf),
            pltpu.VMEM((2, m_per, dh), bf),
            pltpu.VMEM((m_per, dh), bf),
            pltpu.VMEM((m_per, dh), bf),
            pltpu.VMEM((m_per, dh), bf),
            pltpu.VMEM((m_per, dh), bf),
            pltpu.VMEM((m_per, dh), bf),
            pltpu.VMEM((m_per, dh), bf),
            pltpu.SemaphoreType.DMA((4,)),
            pltpu.SemaphoreType.DMA((4,)),
        ],
        compiler_params=pltpu.CompilerParams(collective_id=0),
    )(x, g)


# device time: 4364 ns/iter; 6.7390x vs baseline; 4.3547x over previous
import jax
import jax.numpy as jnp
from jax import lax
from jax.experimental import pallas as pl
from jax.experimental.pallas import tpu as pltpu

N_DEV = 4


def kernel(partial, gamma):
    x = partial.reshape(partial.shape[1], partial.shape[2])
    g = gamma.reshape(1, -1)
    m_tot, d = x.shape
    m_per = m_tot // N_DEV
    dh = d // 2

    def body(
        x_ref, g_ref, out_ref,
        send_a1, send_b1, recv_a1, recv_b1,
        send_a2, send_b2, recv_a2, recv_b2,
        loc_a2, loc_b2,
        send_sems, recv_sems,
    ):
        my = lax.axis_index("i")
        q1 = my ^ 1
        q3 = my ^ 3
        c1 = my ^ 1
        c2 = my ^ 2
        c3 = my ^ 3

        def row(c):
            return pl.ds(c * m_per, m_per)

        def copy(src, dst, sem, target):
            return pltpu.make_async_remote_copy(
                src_ref=src, dst_ref=dst,
                send_sem=send_sems.at[sem], recv_sem=recv_sems.at[sem],
                device_id=(target,), device_id_type=pl.DeviceIdType.MESH,
            )

        send_a1[0, :, :] = x_ref[row(c2), :dh].astype(jnp.bfloat16)
        send_b1[0, :, :] = x_ref[row(c2), dh:].astype(jnp.bfloat16)

        barrier_sem = pltpu.get_barrier_semaphore()
        for nbr in (q1, q3):
            pl.semaphore_signal(
                barrier_sem, inc=1,
                device_id=(nbr,), device_id_type=pl.DeviceIdType.MESH,
            )
        pl.semaphore_wait(barrier_sem, 2)

        rdma_a1u = copy(send_a1.at[0], recv_a1.at[0], 0, q1)
        rdma_b1u = copy(send_b1.at[0], recv_b1.at[0], 1, q3)
        rdma_a1u.start()
        rdma_b1u.start()

        send_a1[1, :, :] = x_ref[row(c1), :dh].astype(jnp.bfloat16)
        send_b1[1, :, :] = x_ref[row(c3), dh:].astype(jnp.bfloat16)
        rdma_a1l = copy(send_a1.at[1], recv_a1.at[1], 2, q1)
        rdma_b1l = copy(send_b1.at[1], recv_b1.at[1], 3, q3)
        rdma_a1l.start()
        rdma_b1l.start()

        loc_a2[:, :] = x_ref[row(c3), :dh].astype(jnp.bfloat16)
        loc_b2[:, :] = x_ref[row(c1), dh:].astype(jnp.bfloat16)

        rdma_a1u.wait_recv()
        send_a2[:, :] = loc_a2[:, :] + recv_a1[0, :, :]
        rdma_a2 = copy(send_a2, recv_a2, 4, q3)
        rdma_a2.start()

        rdma_b1u.wait_recv()
        send_b2[:, :] = loc_b2[:, :] + recv_b1[0, :, :]
        rdma_b2 = copy(send_b2, recv_b2, 5, q1)
        rdma_b2.start()

        rdma_a1l.wait_recv()
        rdma_b1l.wait_recv()
        rdma_a2.wait_recv()
        rdma_b2.wait_recv()
        y_a = (
            x_ref[row(my), :dh]
            + recv_a1[1, :, :].astype(jnp.float32)
            + recv_a2[:, :].astype(jnp.float32)
        )
        y_b = (
            x_ref[row(my), dh:]
            + recv_b1[1, :, :].astype(jnp.float32)
            + recv_b2[:, :].astype(jnp.float32)
        )
        ss = jnp.sum(y_a * y_a, axis=-1, keepdims=True) + jnp.sum(
            y_b * y_b, axis=-1, keepdims=True
        )
        inv_rms = lax.rsqrt(ss / d + 1e-6)
        out_ref[:, :dh] = y_a * inv_rms * g_ref[0, :dh]
        out_ref[:, dh:] = y_b * inv_rms * g_ref[0, dh:]

        for r in (rdma_a1u, rdma_b1u, rdma_a1l, rdma_b1l, rdma_a2, rdma_b2):
            r.wait_send()

    bf = jnp.bfloat16
    return pl.pallas_call(
        body,
        out_shape=jax.ShapeDtypeStruct((m_per, d), jnp.float32),
        in_specs=[
            pl.BlockSpec(memory_space=pltpu.VMEM),
            pl.BlockSpec(memory_space=pltpu.VMEM),
        ],
        out_specs=pl.BlockSpec(memory_space=pltpu.VMEM),
        scratch_shapes=[
            pltpu.VMEM((2, m_per, dh), bf),
            pltpu.VMEM((2, m_per, dh), bf),
            pltpu.VMEM((2, m_per, dh), bf),
            pltpu.VMEM((2, m_per, dh), bf),
            pltpu.VMEM((m_per, dh), bf),
            pltpu.VMEM((m_per, dh), bf),
            pltpu.VMEM((m_per, dh), bf),
            pltpu.VMEM((m_per, dh), bf),
            pltpu.VMEM((m_per, dh), bf),
            pltpu.VMEM((m_per, dh), bf),
            pltpu.SemaphoreType.DMA((6,)),
            pltpu.SemaphoreType.DMA((6,)),
        ],
        compiler_params=pltpu.CompilerParams(collective_id=0),
    )(x, g)
